# baseline (speedup 1.0000x reference)
import jax
import jax.numpy as jnp
from jax import lax
from jax.experimental import pallas as pl
from jax.experimental.pallas import tpu as pltpu
from jax.experimental.pallas import tpu_sc as plsc

HIDDEN = 64
SEQ = 200
BATCH = 4096
NUM_WORKERS = 32
BPW = BATCH // NUM_WORKERS
HT = HIDDEN // 8


def _splat(x):
    return jnp.full((16,), x, dtype=jnp.int32)


def _start_gather(wtab_hbm, idx_all, rows, semg, s):
    pltpu.async_copy(wtab_hbm.at[idx_all.at[s]], rows, semg)


def _drain(src_hbm, dst, sem):
    pltpu.make_async_copy(src_hbm, dst, sem).wait()


def _combine(rows, stg, pos_v, s):
    iota = lax.iota(jnp.int32, 16)

    def h_body(h, carry):
        pos_b = plsc.load_gather(pos_v, [_splat(s), _splat(h)])
        for k in range(BPW // 16):
            v = plsc.load_gather(rows, [iota + (k * 16), _splat(h)])
            stg[h, pl.ds(k * 16, 16)] = jnp.maximum(v + pos_b, 0.0)
        return carry

    lax.fori_loop(0, HIDDEN, h_body, 0)


def _sc_body(idx_hbm, wtab_hbm, ptab_hbm, out_hbm,
             idx_all, rows, stg, pos_v, semg0, semg1, semo0, semo1):
    nc = 2
    wid = lax.axis_index("s") * nc + lax.axis_index("c")

    pltpu.sync_copy(ptab_hbm, pos_v)
    pltpu.sync_copy(idx_hbm.at[:, pl.ds(wid * BPW, BPW)], idx_all)

    semg = (semg0, semg1)
    semo = (semo0, semo1)

    _start_gather(wtab_hbm, idx_all, rows.at[0], semg0, 0)

    def pair_body(t, carry):
        s0 = 2 * t
        for par in range(2):
            s = s0 + par
            nxt = s + 1
            nxt = jnp.where(nxt >= SEQ, 0, nxt)
            _start_gather(wtab_hbm, idx_all, rows.at[1 - par],
                          semg[1 - par], nxt)
            _drain(wtab_hbm.at[pl.ds(0, BPW)], rows.at[par], semg[par])
            @pl.when(s >= 2)
            def _():
                for th in range(HT):
                    _drain(out_hbm.at[0, 0, 0],
                           stg.at[par].at[pl.ds(th * 8, 8)], semo[par])
            _combine(rows.at[par], stg.at[par], pos_v, s)
            for th in range(HT):
                pltpu.async_copy(stg.at[par].at[pl.ds(th * 8, 8)],
                                 out_hbm.at[s, th, wid], semo[par])
        return carry

    lax.fori_loop(0, SEQ // 2, pair_body, 0)

    _drain(wtab_hbm.at[pl.ds(0, BPW)], rows.at[0], semg0)
    for par in range(2):
        for th in range(HT):
            _drain(out_hbm.at[0, 0, 0],
                   stg.at[par].at[pl.ds(th * 8, 8)], semo[par])


@jax.jit
def kernel(input_seq, word_table, pos_table):
    batch, seq = input_seq.shape
    idx_t = jnp.swapaxes(input_seq, 0, 1).astype(jnp.int32)

    mesh = plsc.VectorSubcoreMesh(core_axis_name="c", subcore_axis_name="s")
    run = pl.kernel(
        _sc_body,
        out_type=jax.ShapeDtypeStruct((SEQ, HT, NUM_WORKERS, 8, 128),
                                      jnp.float32),
        mesh=mesh,
        scratch_types=[
            pltpu.VMEM((SEQ, BPW), jnp.int32),
            pltpu.VMEM((2, BPW, HIDDEN), jnp.float32),
            pltpu.VMEM((2, HIDDEN, 128), jnp.float32),
            pltpu.VMEM((SEQ, HIDDEN), jnp.float32),
            pltpu.SemaphoreType.DMA,
            pltpu.SemaphoreType.DMA,
            pltpu.SemaphoreType.DMA,
            pltpu.SemaphoreType.DMA,
        ],
        compiler_params=pltpu.CompilerParams(use_tc_tiling_on_sc=False,
                                             needs_layout_passes=False),
    )
    out5d = run(idx_t, word_table, pos_table)
    return out5d.transpose(2, 4, 0, 1, 3).reshape(batch, seq, HIDDEN)
